# baseline (speedup 1.0000x reference)
import jax
import jax.numpy as jnp
from jax import lax
from jax.experimental import pallas as pl
from jax.experimental.pallas import tpu as pltpu, tpu_sc as plsc

N_USERS = 5000
N_ITEMS = 5000
N = N_USERS + N_ITEMS
D = 256
DH = 128
N_HOPS = 3
NNZ = 160000
NC = 2
NS = 16
L = 16

E = 64
EPT = 10240
NNZ_PAD = EPT * NS
NCHUNK = EPT // E
CHUNKS_TOT = NNZ_PAD // E

R = 16
ROWS_BIG = 640
ROWS_LAST = N - (NS - 1) * ROWS_BIG
COEF_PAD = ROWS_BIG * NS


def _scale_rows(dst, src, nrows, coef_ref, crow, cbase):
    zeros16 = lax.broadcasted_iota(jnp.int32, (L,), 0) * 0

    def body(g, _):
        c16 = coef_ref[crow, pl.ds(cbase + g * L, L)]

        def lanes(q, _):
            for k in range(4):
                e = q * 4 + k
                splat = c16.at[zeros16 + e].get(mode="promise_in_bounds")
                row = g * L + e
                for j in range(DH // L):
                    sl = pl.ds(j * L, L)
                    dst[row, sl] = src[row, sl] * splat
            return 0
        lax.fori_loop(0, L // 4, lanes, 0)
        return 0
    lax.fori_loop(0, nrows // L, body, 0)


def _graph_conv(agg0, coefs, packed, valsf, zblk):
    mesh = plsc.VectorSubcoreMesh(core_axis_name="c", subcore_axis_name="s",
                                  num_cores=NC, num_subcores=NS)

    def body(agg0_hbm, coef_hbm, packed_hbm, valsf_hbm, zblk_hbm,
             out_hbm, scr1_hbm, scr2_hbm,
             acc, idx0, idx1, idx2, idx3, vls0, vls1, vls2, vls3,
             gbuf0, gbuf1, sbuf0, sbuf1,
             rows0, rows1, coef_v, stage_v,
             isem0, isem1, isem2, isem3, gsem0, gsem1, ssem0, ssem1):
        c = lax.axis_index("c")
        s = lax.axis_index("s")
        idx = [idx0, idx1, idx2, idx3]
        vls = [vls0, vls1, vls2, vls3]
        isem = [isem0, isem1, isem2, isem3]
        gbuf = [gbuf0, gbuf1]
        sbuf = [sbuf0, sbuf1]
        rows_s = [rows0, rows1]
        gsem = [gsem0, gsem1]
        ssem = [ssem0, ssem1]

        cbase = s * NCHUNK
        row_base = s * ROWS_BIG
        nrows = jnp.where(s < NS - 1, ROWS_BIG, ROWS_LAST)
        nrc = nrows // R

        def zero_my_rows():
            pltpu.sync_copy(zblk_hbm.at[pl.ds(0, ROWS_LAST)],
                            acc.at[pl.ds(row_base, ROWS_LAST)])

            @pl.when(s < NS - 1)
            def _():
                pltpu.sync_copy(
                    zblk_hbm.at[pl.ds(0, ROWS_BIG - ROWS_LAST)],
                    acc.at[pl.ds(row_base + ROWS_LAST, ROWS_BIG - ROWS_LAST)])

        def out_scaled(h, src3, stage):
            pltpu.sync_copy(
                coef_hbm.at[pl.ds(h, 1), pl.ds(row_base, ROWS_BIG)], coef_v)

            def body_(j, _):
                rb = row_base + j * R
                if src3 is None:
                    pltpu.sync_copy(acc.at[pl.ds(rb, R)], stage)
                else:
                    pltpu.sync_copy(src3.at[c, pl.ds(rb, R)], stage)
                _scale_rows(stage, stage, R, coef_v, 0, j * R)
                pltpu.sync_copy(
                    stage, out_hbm.at[pl.ds(rb, R), h, pl.ds(c * DH, DH)])
                return 0
            lax.fori_loop(0, nrc, body_, 0)

        zero_my_rows()
        out_scaled(0, agg0_hbm, stage_v)
        plsc.subcore_barrier()

        hop_srcs = [agg0_hbm, scr1_hbm, scr2_hbm]
        hop_raws = [scr1_hbm, scr2_hbm, None]

        for h in range(1, N_HOPS + 1):
            src = hop_srcs[h - 1]
            raw = hop_raws[h - 1]

            def gather_start(chunk, bi, b, src=src):
                pltpu.async_copy(
                    src.at[c].at[idx[bi].at[1]], gbuf[b], gsem[b])

            def gather_wait(bi, b, src=src):
                pltpu.make_async_copy(
                    src.at[c].at[idx[bi].at[1]], gbuf[b], gsem[b]).wait()

            for p in range(2):
                pltpu.sync_copy(packed_hbm.at[cbase + p], idx[p])
                pltpu.sync_copy(valsf_hbm.at[pl.ds(cbase + p, 1)], vls[p])
                gather_start(cbase + p, p, p)

            def edge_quad(o, _):
                for cc in range(4):
                    chunk = o * 4 + cc
                    b = cc % 2
                    bi = cc
                    bi2 = (cc + 2) % 4

                    @pl.when(chunk >= 2)
                    def _():
                        pltpu.make_async_copy(
                            sbuf[b], acc.at[rows_s[b]], ssem[b]).wait()

                    @pl.when(chunk + 2 < NCHUNK)
                    def _():
                        pltpu.async_copy(packed_hbm.at[cbase + chunk + 2],
                                         idx[bi2], isem[bi2])
                        pltpu.async_copy(
                            valsf_hbm.at[pl.ds(cbase + chunk + 2, 1)],
                            vls[bi2], isem[bi2])

                    gather_wait(bi, b)
                    _scale_rows(sbuf[b], gbuf[b], E, vls[bi], 0, 0)
                    for k in range(E // L):
                        sl = pl.ds(k * L, L)
                        rows_s[b][sl] = idx[bi][0, sl]
                    pltpu.async_copy(sbuf[b], acc.at[rows_s[b]], ssem[b],
                                     add=True)

                    @pl.when(chunk + 2 < NCHUNK)
                    def _():
                        pltpu.make_async_copy(
                            packed_hbm.at[cbase + chunk + 2],
                            idx[bi2], isem[bi2]).wait()
                        pltpu.make_async_copy(
                            valsf_hbm.at[pl.ds(cbase + chunk + 2, 1)],
                            vls[bi2], isem[bi2]).wait()
                        gather_start(cbase + chunk + 2, bi2, b)
                return 0
            lax.fori_loop(0, NCHUNK // 4, edge_quad, 0)

            for b in range(2):
                pltpu.make_async_copy(
                    sbuf[b], acc.at[rows_s[b]], ssem[b]).wait()

            plsc.subcore_barrier()

            if raw is not None:
                pltpu.sync_copy(acc.at[pl.ds(row_base, ROWS_LAST)],
                                raw.at[c, pl.ds(row_base, ROWS_LAST)])

                @pl.when(s < NS - 1)
                def _():
                    pltpu.sync_copy(
                        acc.at[pl.ds(row_base + ROWS_LAST,
                                     ROWS_BIG - ROWS_LAST)],
                        raw.at[c, pl.ds(row_base + ROWS_LAST,
                                        ROWS_BIG - ROWS_LAST)])

            out_scaled(h, None, stage_v)

            if raw is not None:
                zero_my_rows()

            plsc.subcore_barrier()

    f = pl.kernel(
        body,
        out_type=(
            jax.ShapeDtypeStruct((N, N_HOPS + 1, D), jnp.float32),
            jax.ShapeDtypeStruct((NC, N, DH), jnp.float32),
            jax.ShapeDtypeStruct((NC, N, DH), jnp.float32),
        ),
        mesh=mesh,
        scratch_types=[
            pltpu.VMEM_SHARED((N, DH), jnp.float32),
            pltpu.VMEM((2, E), jnp.int32),
            pltpu.VMEM((2, E), jnp.int32),
            pltpu.VMEM((2, E), jnp.int32),
            pltpu.VMEM((2, E), jnp.int32),
            pltpu.VMEM((1, E), jnp.float32),
            pltpu.VMEM((1, E), jnp.float32),
            pltpu.VMEM((1, E), jnp.float32),
            pltpu.VMEM((1, E), jnp.float32),
            pltpu.VMEM((E, DH), jnp.float32),
            pltpu.VMEM((E, DH), jnp.float32),
            pltpu.VMEM((E, DH), jnp.float32),
            pltpu.VMEM((E, DH), jnp.float32),
            pltpu.VMEM((E,), jnp.int32),
            pltpu.VMEM((E,), jnp.int32),
            pltpu.VMEM((1, ROWS_BIG), jnp.float32),
            pltpu.VMEM((R, DH), jnp.float32),
            pltpu.SemaphoreType.DMA,
            pltpu.SemaphoreType.DMA,
            pltpu.SemaphoreType.DMA,
            pltpu.SemaphoreType.DMA,
            pltpu.SemaphoreType.DMA,
            pltpu.SemaphoreType.DMA,
            pltpu.SemaphoreType.DMA,
            pltpu.SemaphoreType.DMA,
        ],
    )
    out, _, _ = f(agg0, coefs, packed, valsf, zblk)
    return out


def kernel(user_embed, item_embed, user_t, item_t, edge_rows, edge_cols, edge_vals):
    all_embed = jnp.concatenate([user_embed, item_embed], axis=0)
    agg0 = all_embed.reshape(N, NC, DH).transpose(1, 0, 2)

    t = jnp.concatenate([user_t[:, 0], item_t[:, 0]])
    decay = 1.0 - t
    coefs = jnp.stack([t, t * decay, t * decay**2, t * decay**3])
    coefs = jnp.pad(coefs, ((0, 0), (0, COEF_PAD - N)))

    pad = NNZ_PAD - NNZ
    rows_p = jnp.pad(edge_rows, (0, pad))
    cols_p = jnp.pad(edge_cols, (0, pad))
    packed = jnp.stack([rows_p.reshape(CHUNKS_TOT, E),
                        cols_p.reshape(CHUNKS_TOT, E)], axis=1)
    valsf = jnp.pad(edge_vals, (0, pad)).reshape(CHUNKS_TOT, E)

    zblk = jnp.zeros((ROWS_BIG, DH), jnp.float32)

    out = _graph_conv(agg0, coefs, packed, valsf, zblk)
    return out[:N_USERS], out[N_USERS:]


# speedup vs baseline: 1.4529x; 1.4529x over previous
import jax
import jax.numpy as jnp
from jax import lax
from jax.experimental import pallas as pl
from jax.experimental.pallas import tpu as pltpu, tpu_sc as plsc

N_USERS = 5000
N_ITEMS = 5000
N = N_USERS + N_ITEMS
D = 256
DH = 128
N_HOPS = 3
NNZ = 160000
NC = 2
NS = 16
L = 16

E = 64
EPT = 10240
NNZ_PAD = EPT * NS
NCHUNK = EPT // E
SUP = 8
SUPE = SUP * E
NSUP = NCHUNK // SUP
TOT_SC = NNZ_PAD // SUPE

R = 16
ROWS_BIG = 640
ROWS_LAST = N - (NS - 1) * ROWS_BIG
COEF_PAD = ROWS_BIG * NS


def _scale_rows(dst, src, nrows, coef_ref, cbase):
    zeros16 = lax.broadcasted_iota(jnp.int32, (L,), 0) * 0

    def body(g, _):
        c16 = coef_ref[0, pl.ds(cbase + g * L, L)]

        def lanes(q, _):
            for k in range(8):
                e = q * 8 + k
                splat = c16.at[zeros16 + e].get(mode="promise_in_bounds")
                row = g * L + e
                for j in range(DH // L):
                    sl = pl.ds(j * L, L)
                    dst[row, sl] = src[row, sl] * splat
            return 0
        lax.fori_loop(0, L // 8, lanes, 0)
        return 0
    lax.fori_loop(0, nrows // L, body, 0)


def _graph_conv(agg0, coefs, packed, valsf, zblk):
    mesh = plsc.VectorSubcoreMesh(core_axis_name="c", subcore_axis_name="s",
                                  num_cores=NC, num_subcores=NS)

    def body(agg0_hbm, coef_hbm, packed_hbm, valsf_hbm, zblk_hbm,
             out_hbm, scr1_hbm, scr2_hbm,
             acc, idx_sc, vls_sc, gbuf0, gbuf1, gbuf2, gbuf3,
             rows0, rows1, rows2, rows3, coef_v, stage_v,
             gsem0, gsem1, gsem2, gsem3, ssem0, ssem1, ssem2, ssem3, csem):
        c = lax.axis_index("c")
        s = lax.axis_index("s")
        gbuf = [gbuf0, gbuf1, gbuf2, gbuf3]
        rows_s = [rows0, rows1, rows2, rows3]
        gsem = [gsem0, gsem1, gsem2, gsem3]
        ssem = [ssem0, ssem1, ssem2, ssem3]

        scbase = s * NSUP
        row_base = s * ROWS_BIG
        nrows = jnp.where(s < NS - 1, ROWS_BIG, ROWS_LAST)
        nrc = nrows // R

        def zero_my_rows():
            pltpu.sync_copy(zblk_hbm.at[pl.ds(0, ROWS_LAST)],
                            acc.at[pl.ds(row_base, ROWS_LAST)])

            @pl.when(s < NS - 1)
            def _():
                pltpu.sync_copy(
                    zblk_hbm.at[pl.ds(0, ROWS_BIG - ROWS_LAST)],
                    acc.at[pl.ds(row_base + ROWS_LAST, ROWS_BIG - ROWS_LAST)])

        def out_scaled(h, src3, stage):
            pltpu.sync_copy(
                coef_hbm.at[pl.ds(h, 1), pl.ds(row_base, ROWS_BIG)], coef_v)

            def body_(j, _):
                rb = row_base + j * R
                if src3 is None:
                    pltpu.sync_copy(acc.at[pl.ds(rb, R)], stage)
                else:
                    pltpu.sync_copy(src3.at[c, pl.ds(rb, R)], stage)
                _scale_rows(stage, stage, R, coef_v, j * R)
                pltpu.sync_copy(
                    stage, out_hbm.at[pl.ds(rb, R), h, pl.ds(c * DH, DH)])
                return 0
            lax.fori_loop(0, nrc, body_, 0)

        zero_my_rows()
        out_scaled(0, agg0_hbm, stage_v)
        plsc.subcore_barrier()

        hop_srcs = [agg0_hbm, scr1_hbm, scr2_hbm]
        hop_raws = [scr1_hbm, scr2_hbm, None]

        def sc_load_start(t):
            off = (t % 2) * SUPE
            pltpu.async_copy(packed_hbm.at[scbase + t],
                             idx_sc.at[:, pl.ds(off, SUPE)], csem)
            pltpu.async_copy(valsf_hbm.at[pl.ds(scbase + t, 1)],
                             vls_sc.at[:, pl.ds(off, SUPE)], csem)

        def sc_load_wait(t):
            off = (t % 2) * SUPE
            pltpu.make_async_copy(packed_hbm.at[scbase + t],
                                  idx_sc.at[:, pl.ds(off, SUPE)], csem).wait()
            pltpu.make_async_copy(valsf_hbm.at[pl.ds(scbase + t, 1)],
                                  vls_sc.at[:, pl.ds(off, SUPE)], csem).wait()

        for h in range(1, N_HOPS + 1):
            src = hop_srcs[h - 1]
            raw = hop_raws[h - 1]

            def gather_start(eoff, b, src=src):
                pltpu.async_copy(
                    src.at[c].at[idx_sc.at[1, pl.ds(eoff, E)]],
                    gbuf[b], gsem[b])

            def gather_wait(eoff, b, src=src):
                pltpu.make_async_copy(
                    src.at[c].at[idx_sc.at[1, pl.ds(eoff, E)]],
                    gbuf[b], gsem[b]).wait()

            sc_load_start(0)
            sc_load_wait(0)
            sc_load_start(1)
            for p in range(4):
                gather_start(p * E, p)

            def super_body(t, _):
                u = t % 2
                off0 = u * SUPE

                for lc in range(SUP):
                    chunk = t * SUP + lc
                    b = lc % 4
                    eoff = off0 + lc * E

                    gather_wait(eoff, b)
                    _scale_rows(gbuf[b], gbuf[b], E, vls_sc, eoff)
                    for k in range(E // L):
                        sl = pl.ds(k * L, L)
                        rows_s[b][sl] = idx_sc[0, pl.ds(eoff + k * L, L)]
                    pltpu.async_copy(gbuf[b], acc.at[rows_s[b]], ssem[b],
                                     add=True)

                    if lc == 5:
                        @pl.when(t + 1 < NSUP)
                        def _():
                            sc_load_wait(t + 1)

                    bp = (lc + 3) % 4
                    if lc < 5:
                        eoff3 = off0 + (lc + 3) * E
                    else:
                        eoff3 = (SUPE - off0) + (lc - 5) * E

                    @pl.when((chunk >= 1) & (chunk + 3 < NCHUNK))
                    def _(bp=bp, eoff3=eoff3):
                        pltpu.make_async_copy(
                            gbuf[bp], acc.at[rows_s[bp]], ssem[bp]).wait()
                        gather_start(eoff3, bp)

                    if lc == SUP - 1:
                        @pl.when(t + 2 < NSUP)
                        def _():
                            sc_load_start(t + 2)
                return 0
            lax.fori_loop(0, NSUP, super_body, 0)

            for b in range(4):
                pltpu.make_async_copy(
                    gbuf[b], acc.at[rows_s[b]], ssem[b]).wait()

            plsc.subcore_barrier()

            if raw is not None:
                pltpu.sync_copy(acc.at[pl.ds(row_base, ROWS_LAST)],
                                raw.at[c, pl.ds(row_base, ROWS_LAST)])

                @pl.when(s < NS - 1)
                def _():
                    pltpu.sync_copy(
                        acc.at[pl.ds(row_base + ROWS_LAST,
                                     ROWS_BIG - ROWS_LAST)],
                        raw.at[c, pl.ds(row_base + ROWS_LAST,
                                        ROWS_BIG - ROWS_LAST)])

            out_scaled(h, None, stage_v)

            if raw is not None:
                zero_my_rows()

            plsc.subcore_barrier()

    f = pl.kernel(
        body,
        out_type=(
            jax.ShapeDtypeStruct((N, N_HOPS + 1, D), jnp.float32),
            jax.ShapeDtypeStruct((NC, N, DH), jnp.float32),
            jax.ShapeDtypeStruct((NC, N, DH), jnp.float32),
        ),
        mesh=mesh,
        scratch_types=[
            pltpu.VMEM_SHARED((N, DH), jnp.float32),
            pltpu.VMEM((2, 2 * SUPE), jnp.int32),
            pltpu.VMEM((1, 2 * SUPE), jnp.float32),
            pltpu.VMEM((E, DH), jnp.float32),
            pltpu.VMEM((E, DH), jnp.float32),
            pltpu.VMEM((E, DH), jnp.float32),
            pltpu.VMEM((E, DH), jnp.float32),
            pltpu.VMEM((E,), jnp.int32),
            pltpu.VMEM((E,), jnp.int32),
            pltpu.VMEM((E,), jnp.int32),
            pltpu.VMEM((E,), jnp.int32),
            pltpu.VMEM((1, ROWS_BIG), jnp.float32),
            pltpu.VMEM((R, DH), jnp.float32),
            pltpu.SemaphoreType.DMA,
            pltpu.SemaphoreType.DMA,
            pltpu.SemaphoreType.DMA,
            pltpu.SemaphoreType.DMA,
            pltpu.SemaphoreType.DMA,
            pltpu.SemaphoreType.DMA,
            pltpu.SemaphoreType.DMA,
            pltpu.SemaphoreType.DMA,
            pltpu.SemaphoreType.DMA,
        ],
    )
    out, _, _ = f(agg0, coefs, packed, valsf, zblk)
    return out


def kernel(user_embed, item_embed, user_t, item_t, edge_rows, edge_cols, edge_vals):
    all_embed = jnp.concatenate([user_embed, item_embed], axis=0)
    agg0 = all_embed.reshape(N, NC, DH).transpose(1, 0, 2)

    t = jnp.concatenate([user_t[:, 0], item_t[:, 0]])
    decay = 1.0 - t
    coefs = jnp.stack([t, t * decay, t * decay**2, t * decay**3])
    coefs = jnp.pad(coefs, ((0, 0), (0, COEF_PAD - N)))

    pad = NNZ_PAD - NNZ
    rows_p = jnp.pad(edge_rows, (0, pad))
    cols_p = jnp.pad(edge_cols, (0, pad))
    packed = jnp.stack([rows_p.reshape(TOT_SC, SUPE),
                        cols_p.reshape(TOT_SC, SUPE)], axis=1)
    valsf = jnp.pad(edge_vals, (0, pad)).reshape(TOT_SC, SUPE)

    zblk = jnp.zeros((ROWS_BIG, DH), jnp.float32)

    out = _graph_conv(agg0, coefs, packed, valsf, zblk)
    return out[:N_USERS], out[N_USERS:]
